# Initial kernel scaffold; baseline (speedup 1.0000x reference)
import jax
import jax.numpy as jnp
from jax.experimental import pallas as pl


def kernel(rd1_w1m, rd1_b1, rd1_w2m, rd1_b2, rd1_wsm, rd1_bs, rd2_w1m, rd2_b1, rd2_w2m, rd2_b2, rd2_wsm, rd2_bs, rd3_w1m, rd3_b1, rd3_w2m, rd3_b2, rd3_wsm, rd3_bs, rd4_w1m, rd4_b1, rd4_w2m, rd4_b2, rd4_wsm, rd4_bs, rd5_w1m, rd5_b1, rd5_w2m, rd5_b2, rd5_wsm, rd5_bs, rd6_w1m, rd6_b1, rd6_w2m, rd6_b2, linear_w, linear_b, proj_w, proj_b, aux1_w, aux1_b, aux2_w, aux2_b, x_src, x_tgt, y):
    raise NotImplementedError("write your pallas kernel here")


# speedup vs baseline: 3.1726x; 3.1726x over previous
import functools

import jax
import jax.numpy as jnp
from jax.experimental import pallas as pl
from jax.experimental.pallas import tpu as pltpu

_VMEM = dict(vmem_limit_bytes=100 * 1024 * 1024)


def _pad_hw(x):
    bn, H, W, C = x.shape
    zc = jnp.zeros((bn, H, 1, C), x.dtype)
    x = jnp.concatenate([zc, x, zc], axis=2)
    zr = jnp.zeros((bn, 1, W + 2, C), x.dtype)
    return jnp.concatenate([zr, x, zr], axis=1)


def _pool2(x):
    bn, H, W, C = x.shape
    x = x.reshape(bn, H // 2, 2, W, C)
    x = x[:, :, 0] + x[:, :, 1]
    x = x.reshape(bn, H // 2, W // 2, 2, C)
    return (x[:, :, :, 0] + x[:, :, :, 1]) * 0.25


def _tap_matmuls(xp_ref, w_ref, *, pre_relu):
    bn, Hp, Wp, Cin = xp_ref.shape
    H, W = Hp - 2, Wp - 2
    acc = None
    for t in range(9):
        dh, dw = divmod(t, 3)
        a = xp_ref[:, dh:dh + H, dw:dw + W, :]
        if pre_relu:
            a = jnp.maximum(a, 0)
        a = a.reshape(bn * H * W, Cin)
        d = jnp.dot(a, w_ref[t], preferred_element_type=jnp.float32)
        acc = d if acc is None else acc + d
    return acc


def _conv1_kernel(xp_ref, w_ref, b_ref, o_ref, *, pre_relu):
    bn, Hp, Wp, _ = xp_ref.shape
    H, W = Hp - 2, Wp - 2
    ct = o_ref.shape[-1]
    acc = _tap_matmuls(xp_ref, w_ref, pre_relu=pre_relu) + b_ref[...]
    out = jnp.maximum(acc, 0.0).astype(jnp.bfloat16).reshape(bn, H, W, ct)
    o_ref[...] = _pad_hw(out)


def _conv2_pool_sc_kernel(hp_ref, xp_ref, w_ref, b_ref, ws_ref, bs_ref, o_ref):
    bn, Hp, Wp, _ = hp_ref.shape
    H, W = Hp - 2, Wp - 2
    ct = o_ref.shape[-1]
    acc = _tap_matmuls(hp_ref, w_ref, pre_relu=False) + b_ref[...]
    h = _pool2(acc.reshape(bn, H, W, ct))
    xin = xp_ref[:, 1:H + 1, 1:W + 1, :].astype(jnp.float32)
    px = _pool2(xin).astype(jnp.bfloat16)
    cin = px.shape[-1]
    sc = jnp.dot(px.reshape(bn * (H // 2) * (W // 2), cin), ws_ref[...],
                 preferred_element_type=jnp.float32) + bs_ref[...]
    out = (h.reshape(-1, ct) + sc).astype(jnp.bfloat16)
    o_ref[...] = _pad_hw(out.reshape(bn, H // 2, W // 2, ct))


def _conv2_id_sum_kernel(hp_ref, xp_ref, w_ref, b_ref, o_ref):
    bn, Hp, Wp, _ = hp_ref.shape
    H, W = Hp - 2, Wp - 2
    ct = o_ref.shape[-1]
    acc = _tap_matmuls(hp_ref, w_ref, pre_relu=False) + b_ref[...]
    xin = xp_ref[:, 1:H + 1, 1:W + 1, :].astype(jnp.float32)
    s = jnp.maximum(acc + xin.reshape(bn * H * W, ct), 0.0)
    o_ref[...] = jnp.sum(s.reshape(bn, H * W, ct), axis=1)


def _head_kernel(x_ref, y_ref, wl_ref, bl_ref, wp_ref, bp_ref,
                 wa1_ref, ba1_ref, wa2_ref, ba2_ref, adv_ref, aux_ref):
    x = x_ref[...]
    adv = jnp.sum(x * wl_ref[...], axis=1, keepdims=True) + bl_ref[...]
    yp = jnp.dot(y_ref[...], wp_ref[...],
                 preferred_element_type=jnp.float32) + bp_ref[...]
    adv = adv + jnp.sum(x * yp, axis=1, keepdims=True)
    adv_ref[...] = adv
    h = jnp.maximum(
        jnp.dot(x, wa1_ref[...], preferred_element_type=jnp.float32)
        + ba1_ref[...], 0.0)
    aux_ref[...] = jnp.dot(h, wa2_ref[...],
                           preferred_element_type=jnp.float32) + ba2_ref[...]


def _conv1(xp, w9, b, cout, *, bi, ct, pre_relu):
    N, Hp, Wp, cin = xp.shape
    grid = (N // bi, cout // ct)
    return pl.pallas_call(
        functools.partial(_conv1_kernel, pre_relu=pre_relu),
        out_shape=jax.ShapeDtypeStruct((N, Hp, Wp, cout), jnp.bfloat16),
        grid=grid,
        in_specs=[
            pl.BlockSpec((bi, Hp, Wp, cin), lambda i, j: (i, 0, 0, 0)),
            pl.BlockSpec((9, cin, ct), lambda i, j: (0, 0, j)),
            pl.BlockSpec((1, ct), lambda i, j: (0, j)),
        ],
        out_specs=pl.BlockSpec((bi, Hp, Wp, ct), lambda i, j: (i, 0, 0, j)),
        compiler_params=pltpu.CompilerParams(
            dimension_semantics=("parallel", "parallel"), **_VMEM),
    )(xp, w9, b)


def _conv2_pool_sc(hp, xp, w9, b, ws, bs, cout, *, bi, ct):
    N, Hp, Wp, c1 = hp.shape
    cin = xp.shape[-1]
    Ho, Wo = (Hp - 2) // 2 + 2, (Wp - 2) // 2 + 2
    grid = (N // bi, cout // ct)
    return pl.pallas_call(
        _conv2_pool_sc_kernel,
        out_shape=jax.ShapeDtypeStruct((N, Ho, Wo, cout), jnp.bfloat16),
        grid=grid,
        in_specs=[
            pl.BlockSpec((bi, Hp, Wp, c1), lambda i, j: (i, 0, 0, 0)),
            pl.BlockSpec((bi, Hp, Wp, cin), lambda i, j: (i, 0, 0, 0)),
            pl.BlockSpec((9, c1, ct), lambda i, j: (0, 0, j)),
            pl.BlockSpec((1, ct), lambda i, j: (0, j)),
            pl.BlockSpec((cin, ct), lambda i, j: (0, j)),
            pl.BlockSpec((1, ct), lambda i, j: (0, j)),
        ],
        out_specs=pl.BlockSpec((bi, Ho, Wo, ct), lambda i, j: (i, 0, 0, j)),
        compiler_params=pltpu.CompilerParams(
            dimension_semantics=("parallel", "parallel"), **_VMEM),
    )(hp, xp, w9, b, ws, bs)


def _conv2_id_sum(hp, xp, w9, b, cout, *, bi, ct):
    N, Hp, Wp, c1 = hp.shape
    grid = (N // bi, cout // ct)
    return pl.pallas_call(
        _conv2_id_sum_kernel,
        out_shape=jax.ShapeDtypeStruct((N, cout), jnp.float32),
        grid=grid,
        in_specs=[
            pl.BlockSpec((bi, Hp, Wp, c1), lambda i, j: (i, 0, 0, 0)),
            pl.BlockSpec((bi, Hp, Wp, ct), lambda i, j: (i, 0, 0, j)),
            pl.BlockSpec((9, c1, ct), lambda i, j: (0, 0, j)),
            pl.BlockSpec((1, ct), lambda i, j: (0, j)),
        ],
        out_specs=pl.BlockSpec((bi, ct), lambda i, j: (i, j)),
        compiler_params=pltpu.CompilerParams(
            dimension_semantics=("parallel", "parallel"), **_VMEM),
    )(hp, xp, w9, b)


def _w9(wm, cin):
    return wm.reshape(9, cin, wm.shape[-1])


def _row(b):
    return b.reshape(1, -1).astype(jnp.float32)


def kernel(rd1_w1m, rd1_b1, rd1_w2m, rd1_b2, rd1_wsm, rd1_bs,
           rd2_w1m, rd2_b1, rd2_w2m, rd2_b2, rd2_wsm, rd2_bs,
           rd3_w1m, rd3_b1, rd3_w2m, rd3_b2, rd3_wsm, rd3_bs,
           rd4_w1m, rd4_b1, rd4_w2m, rd4_b2, rd4_wsm, rd4_bs,
           rd5_w1m, rd5_b1, rd5_w2m, rd5_b2, rd5_wsm, rd5_bs,
           rd6_w1m, rd6_b1, rd6_w2m, rd6_b2,
           linear_w, linear_b, proj_w, proj_b,
           aux1_w, aux1_b, aux2_w, aux2_b,
           x_src, x_tgt, y):
    B = x_src.shape[0]
    x = jnp.concatenate([x_src, x_tgt], axis=0)
    x = jnp.transpose(x, (0, 2, 3, 1)).astype(jnp.bfloat16)
    xp0 = jnp.pad(x, ((0, 0), (1, 1), (1, 1), (0, 0)))

    h = _conv1(xp0, _w9(rd1_w1m, 3), _row(rd1_b1), 64,
               bi=2, ct=64, pre_relu=False)
    o1 = _conv2_pool_sc(h, xp0, _w9(rd1_w2m, 64), _row(rd1_b2),
                        rd1_wsm, _row(rd1_bs), 64, bi=2, ct=64)

    h = _conv1(o1, _w9(rd2_w1m, 64), _row(rd2_b1), 128,
               bi=8, ct=128, pre_relu=True)
    o2 = _conv2_pool_sc(h, o1, _w9(rd2_w2m, 128), _row(rd2_b2),
                        rd2_wsm, _row(rd2_bs), 128, bi=8, ct=128)

    h = _conv1(o2, _w9(rd3_w1m, 128), _row(rd3_b1), 256,
               bi=16, ct=256, pre_relu=True)
    o3 = _conv2_pool_sc(h, o2, _w9(rd3_w2m, 256), _row(rd3_b2),
                        rd3_wsm, _row(rd3_bs), 256, bi=16, ct=256)

    d = o3[B:] - o3[:B]

    h = _conv1(d, _w9(rd4_w1m, 256), _row(rd4_b1), 512,
               bi=16, ct=256, pre_relu=True)
    o4 = _conv2_pool_sc(h, d, _w9(rd4_w2m, 512), _row(rd4_b2),
                        rd4_wsm, _row(rd4_bs), 512, bi=16, ct=256)

    h = _conv1(o4, _w9(rd5_w1m, 512), _row(rd5_b1), 1024,
               bi=16, ct=256, pre_relu=True)
    o5 = _conv2_pool_sc(h, o4, _w9(rd5_w2m, 1024), _row(rd5_b2),
                        rd5_wsm, _row(rd5_bs), 1024, bi=16, ct=256)

    h = _conv1(o5, _w9(rd6_w1m, 1024), _row(rd6_b1), 1024,
               bi=32, ct=256, pre_relu=True)
    xpool = _conv2_id_sum(h, o5, _w9(rd6_w2m, 1024), _row(rd6_b2), 1024,
                          bi=32, ct=256)

    adv, aux = pl.pallas_call(
        _head_kernel,
        out_shape=(jax.ShapeDtypeStruct((B, 1), jnp.float32),
                   jax.ShapeDtypeStruct((B, aux2_w.shape[1]), jnp.float32)),
        compiler_params=pltpu.CompilerParams(**_VMEM),
    )(xpool, y, linear_w, linear_b, proj_w, proj_b,
      aux1_w, aux1_b, aux2_w, aux2_b)
    return adv, aux
